# baseline (speedup 1.0000x reference)
import functools

import jax
import jax.numpy as jnp
from jax import lax
from jax.experimental import pallas as pl
from jax.experimental.pallas import tpu as pltpu
from jax.experimental.pallas import tpu_sc as plsc

BATCH_N = 16384
HIST_N = 50
DIM_N = 32
VOCAB_N = 1000000

NUM_CORES = 2
NUM_SUBCORES = 16
NUM_WORKERS = NUM_CORES * NUM_SUBCORES


def _build(batch=BATCH_N, hist=HIST_N, dim=DIM_N, rows_per_chunk=32,
           interpret=False):
    assert batch % NUM_WORKERS == 0
    b_per_w = batch // NUM_WORKERS
    assert b_per_w % rows_per_chunk == 0
    chunks = b_per_w // rows_per_chunk
    assert hist <= 128
    rows_buf = rows_per_chunk * hist

    mesh = plsc.VectorSubcoreMesh(
        core_axis_name="c", subcore_axis_name="s",
        num_cores=NUM_CORES, num_subcores=NUM_SUBCORES)

    @functools.partial(
        pl.kernel,
        out_type=jax.ShapeDtypeStruct((batch, dim), jnp.float32),
        mesh=mesh,
        scratch_types=[
            pltpu.VMEM((2, rows_per_chunk, hist), jnp.int32),
            pltpu.VMEM((2, rows_buf, dim), jnp.bfloat16),
            pltpu.VMEM((2, rows_per_chunk, dim), jnp.float32),
            pltpu.SemaphoreType.DMA,
            pltpu.SemaphoreType.DMA,
        ],
        compiler_params=pltpu.CompilerParams(
            use_tc_tiling_on_sc=False, needs_layout_passes=False),
        interpret=interpret,
    )
    def _sc_kernel(idx_hbm, table_hbm, out_hbm, idx_v, rows_v, out_v,
                   sem0, sem1):
        sems = (sem0, sem1)
        wid = lax.axis_index("s") * NUM_CORES + lax.axis_index("c")
        bbase = wid * b_per_w

        def fire(c, b):
            pltpu.sync_copy(
                idx_hbm.at[pl.ds(bbase + c * rows_per_chunk,
                                 rows_per_chunk), :],
                idx_v.at[b])
            return [
                pltpu.async_copy(
                    table_hbm.at[idx_v.at[b, j]],
                    rows_v.at[b, pl.ds(j * hist, hist)],
                    sems[b])
                for j in range(rows_per_chunk)
            ]

        ii = lax.broadcasted_iota(jnp.int32, (16,), 0)

        def accumulate(c, b):
            def body(r, carry):
                base = r * hist
                ae, ao = plsc.unpack(rows_v[b, base, :],
                                     format=plsc.PackFormat.INTERLEAVED)
                for l in range(1, hist):
                    e, o = plsc.unpack(rows_v[b, base + l, :],
                                       format=plsc.PackFormat.INTERLEAVED)
                    ae = ae + e
                    ao = ao + o
                rvec = jnp.zeros((16,), jnp.int32) + r
                plsc.store_scatter(out_v.at[b], [rvec, ii * 2], ae)
                plsc.store_scatter(out_v.at[b], [rvec, ii * 2 + 1], ao)
                return carry
            lax.fori_loop(0, rows_per_chunk, body, 0)
            pltpu.sync_copy(
                out_v.at[b],
                out_hbm.at[pl.ds(bbase + c * rows_per_chunk,
                                 rows_per_chunk), :])

        handles = fire(0, 0)
        for c in range(chunks):
            next_handles = fire(c + 1, (c + 1) % 2) if c + 1 < chunks else ()
            for h in handles:
                h.wait()
            accumulate(c, c % 2)
            handles = next_handles

    return _sc_kernel


_gather_impl = _build()


def kernel(inputs, embeddings):
    return _gather_impl(inputs.astype(jnp.int32),
                        embeddings.astype(jnp.bfloat16))


# speedup vs baseline: 1.1927x; 1.1927x over previous
import functools

import jax
import jax.numpy as jnp
from jax import lax
from jax.experimental import pallas as pl
from jax.experimental.pallas import tpu as pltpu
from jax.experimental.pallas import tpu_sc as plsc

BATCH_N = 16384
HIST_N = 50
DIM_N = 32
VOCAB_N = 1000000

NUM_CORES = 2
NUM_SUBCORES = 16
NUM_WORKERS = NUM_CORES * NUM_SUBCORES


def _build(batch=BATCH_N, hist=HIST_N, dim=DIM_N, rows_per_chunk=32,
           interpret=False):
    assert batch % NUM_WORKERS == 0
    b_per_w = batch // NUM_WORKERS
    assert b_per_w % rows_per_chunk == 0
    chunks = b_per_w // rows_per_chunk
    assert hist <= 128
    rows_buf = rows_per_chunk * hist

    mesh = plsc.VectorSubcoreMesh(
        core_axis_name="c", subcore_axis_name="s",
        num_cores=NUM_CORES, num_subcores=NUM_SUBCORES)

    @functools.partial(
        pl.kernel,
        out_type=jax.ShapeDtypeStruct((batch, dim), jnp.float32),
        mesh=mesh,
        scratch_types=[
            pltpu.VMEM((2, rows_per_chunk, hist), jnp.int32),
            pltpu.VMEM((2, rows_buf, dim), jnp.float32),
            pltpu.VMEM((2, rows_per_chunk, dim), jnp.float32),
            pltpu.SemaphoreType.DMA,
            pltpu.SemaphoreType.DMA,
        ],
        compiler_params=pltpu.CompilerParams(use_tc_tiling_on_sc=False),
        interpret=interpret,
    )
    def _sc_kernel(idx_hbm, table_hbm, out_hbm, idx_v, rows_v, out_v,
                   sem0, sem1):
        sems = (sem0, sem1)
        wid = lax.axis_index("s") * NUM_CORES + lax.axis_index("c")
        bbase = wid * b_per_w

        def fire(c, b):
            pltpu.sync_copy(
                idx_hbm.at[pl.ds(bbase + c * rows_per_chunk,
                                 rows_per_chunk), :],
                idx_v.at[b])
            return [
                pltpu.async_copy(
                    table_hbm.at[idx_v.at[b, j]],
                    rows_v.at[b, pl.ds(j * hist, hist)],
                    sems[b])
                for j in range(rows_per_chunk)
            ]

        def accumulate(c, b):
            def body(r, carry):
                base = r * hist
                a0 = rows_v[b, base, pl.ds(0, 16)]
                a1 = rows_v[b, base, pl.ds(16, 16)]
                for l in range(1, hist):
                    a0 = a0 + rows_v[b, base + l, pl.ds(0, 16)]
                    a1 = a1 + rows_v[b, base + l, pl.ds(16, 16)]
                out_v[b, r, pl.ds(0, 16)] = a0
                out_v[b, r, pl.ds(16, 16)] = a1
                return carry
            lax.fori_loop(0, rows_per_chunk, body, 0)
            pltpu.sync_copy(
                out_v.at[b],
                out_hbm.at[pl.ds(bbase + c * rows_per_chunk,
                                 rows_per_chunk), :])

        handles = fire(0, 0)
        for c in range(chunks):
            next_handles = fire(c + 1, (c + 1) % 2) if c + 1 < chunks else ()
            for h in handles:
                h.wait()
            accumulate(c, c % 2)
            handles = next_handles

    return _sc_kernel


_gather_impl = _build()


def kernel(inputs, embeddings):
    return _gather_impl(inputs.astype(jnp.int32), embeddings)
